# Initial kernel scaffold; baseline (speedup 1.0000x reference)
import jax
import jax.numpy as jnp
from jax.experimental import pallas as pl


def kernel(x, w1, b1, w2, b2, w3, b3, gn1_g, gn1_b, gn2_g, gn2_b, gn3_g, gn3_b, fc_w, fc_b):
    raise NotImplementedError("write your pallas kernel here")


# speedup vs baseline: 1.2949x; 1.2949x over previous
import functools

import jax
import jax.numpy as jnp
import numpy as np
from jax.experimental import pallas as pl
from jax.experimental.pallas import tpu as pltpu

_C = 64
_BS = 8
_EPS = 1e-5
_GROUPS = 32

_XP = 255
_P1 = 225
_M2 = 195
_M3 = 48


def _masks_consts():
    m1 = np.zeros((4, _P1, 1), np.float32)
    for a in range(2):
        for b in range(2):
            for u in range(15):
                for v in range(15):
                    r, c = 2 * u + a - 1, 2 * v + b - 1
                    if 0 <= r <= 25 and 0 <= c <= 25:
                        m1[a * 2 + b, u * 15 + v, 0] = 1.0
    m2 = np.zeros((_M2, 1), np.float32)
    for i in range(13):
        for j in range(13):
            m2[i * 15 + j, 0] = 1.0
    m3 = np.zeros((_M3, 1), np.float32)
    for i in range(6):
        for j in range(6):
            m3[i * 8 + j, 0] = 1.0

    def amat(count):
        cg = _C // _GROUPS
        g = np.arange(_C) // cg
        return ((g[:, None] == g[None, :]).astype(np.float32)
                / float(cg * count))

    return (jnp.asarray(m1), jnp.asarray(m2), jnp.asarray(m3),
            jnp.asarray(amat(26 * 26)), jnp.asarray(amat(13 * 13)),
            jnp.asarray(amat(6 * 6)))


def _fused_kernel(x_ref, w1_ref, w2_ref, w3_ref, b1_ref, b2_ref, b3_ref,
                  g1_ref, bb1_ref, g2_ref, bb2_ref, g3_ref, bb3_ref,
                  fcw_ref, fcb_ref, m1_ref, m2_ref, m3_ref,
                  a1_ref, a2_ref, a3_ref, o_ref, y1_ref, z_ref,
                  *, bsz, eps):
    f32 = jnp.float32
    w1 = w1_ref[...]
    b1 = b1_ref[...]

    for a in range(2):
        for bph in range(2):
            cols = []
            for di in range(3):
                for dj in range(3):
                    pi, qi = (a + di) & 1, (a + di) >> 1
                    pj, qj = (bph + dj) & 1, (bph + dj) >> 1
                    st = qi * 15 + qj
                    lane = pi * 2 + pj
                    cols.append(x_ref[:, st:st + _P1, lane:lane + 1])
            xcol = jnp.concatenate(cols, axis=-1).reshape(bsz * _P1, 9)
            acc = jnp.dot(xcol, w1, preferred_element_type=f32)
            y1_ref[:, a * 2 + bph] = acc.reshape(bsz, _P1, _C) + b1

    s = None
    for p in range(4):
        t = jnp.sum(y1_ref[:, p] * m1_ref[p], axis=1)
        s = t if s is None else s + t
    mean1 = jnp.dot(s, a1_ref[...], preferred_element_type=f32)
    v = None
    for p in range(4):
        cen = (y1_ref[:, p] - mean1[:, None, :]) * m1_ref[p]
        t = jnp.sum(cen * cen, axis=1)
        v = t if v is None else v + t
    var1 = jnp.dot(v, a1_ref[...], preferred_element_type=f32)
    sc1 = jax.lax.rsqrt(var1 + eps)
    g1 = g1_ref[...]
    bb1 = bb1_ref[...]
    for p in range(4):
        y = (y1_ref[:, p] - mean1[:, None, :]) * sc1[:, None, :] * g1 + bb1
        y1_ref[:, p] = jnp.maximum(y, 0.0) * m1_ref[p]

    acc2 = None
    for ki in range(4):
        for kj in range(4):
            p = (ki & 1) * 2 + (kj & 1)
            st = (ki >> 1) * 15 + (kj >> 1)
            xt = y1_ref[:, p, st:st + _M2, :].reshape(bsz * _M2, _C)
            d = jnp.dot(xt, w2_ref[ki * 4 + kj], preferred_element_type=f32)
            acc2 = d if acc2 is None else acc2 + d
    raw2 = acc2.reshape(bsz, _M2, _C) + b2_ref[...]
    m2 = m2_ref[...]
    s2 = jnp.sum(raw2 * m2, axis=1)
    mean2 = jnp.dot(s2, a2_ref[...], preferred_element_type=f32)
    cen2 = (raw2 - mean2[:, None, :]) * m2
    var2 = jnp.dot(jnp.sum(cen2 * cen2, axis=1), a2_ref[...],
                   preferred_element_type=f32)
    sc2 = jax.lax.rsqrt(var2 + eps)
    y2 = (raw2 - mean2[:, None, :]) * sc2[:, None, :] * g2_ref[...] \
        + bb2_ref[...]
    y2 = jnp.maximum(y2, 0.0) * m2

    z_ref[...] = jnp.zeros((bsz, 16, 16, _C), f32)
    z_ref[:, 1:14, 1:16, :] = y2.reshape(bsz, 13, 15, _C)
    z6 = z_ref[...].reshape(bsz, 8, 2, 8, 2, _C)
    ph3 = []
    for c in range(2):
        for d in range(2):
            ph3.append(z6[:, :, c, :, d, :].reshape(bsz, 64, _C))

    acc3 = None
    for ki in range(4):
        for kj in range(4):
            ph = ph3[(ki & 1) * 2 + (kj & 1)]
            st = (ki >> 1) * 8 + (kj >> 1)
            xt = ph[:, st:st + _M3, :].reshape(bsz * _M3, _C)
            d = jnp.dot(xt, w3_ref[ki * 4 + kj], preferred_element_type=f32)
            acc3 = d if acc3 is None else acc3 + d
    raw3 = acc3.reshape(bsz, _M3, _C) + b3_ref[...]
    m3 = m3_ref[...]
    s3 = jnp.sum(raw3 * m3, axis=1)
    mean3 = jnp.dot(s3, a3_ref[...], preferred_element_type=f32)
    cen3 = (raw3 - mean3[:, None, :]) * m3
    var3 = jnp.dot(jnp.sum(cen3 * cen3, axis=1), a3_ref[...],
                   preferred_element_type=f32)
    sc3 = jax.lax.rsqrt(var3 + eps)
    y3 = (raw3 - mean3[:, None, :]) * sc3[:, None, :] * g3_ref[...] \
        + bb3_ref[...]
    y3 = jnp.maximum(y3, 0.0) * m3
    pooled = jnp.sum(y3, axis=1) * (1.0 / 36.0)
    o_ref[...] = jnp.dot(pooled, fcw_ref[...],
                         preferred_element_type=f32) + fcb_ref[...]


def _bcast(shape):
    zeros = (0,) * len(shape)
    return pl.BlockSpec(shape, lambda i, _z=zeros: _z)


def kernel(x, w1, b1, w2, b2, w3, b3, gn1_g, gn1_b, gn2_g, gn2_b,
           gn3_g, gn3_b, fc_w, fc_b):
    n = x.shape[0]
    c = _C
    bsz = _BS
    xp = jnp.pad(x[:, 0], ((0, 0), (1, 5), (1, 1)))
    xph = xp.reshape(n, 17, 2, 15, 2).transpose(0, 2, 4, 1, 3)
    xph = xph.reshape(n, 4, _XP).transpose(0, 2, 1)

    w1c = w1.reshape(c, 9).T
    w2t = jnp.transpose(w2, (2, 3, 1, 0)).reshape(16, c, c)
    w3t = jnp.transpose(w3, (2, 3, 1, 0)).reshape(16, c, c)
    n_cls = fc_w.shape[0]
    m1, m2, m3, a1, a2, a3 = _masks_consts()

    out = pl.pallas_call(
        functools.partial(_fused_kernel, bsz=bsz, eps=_EPS),
        grid=(n // bsz,),
        in_specs=[
            pl.BlockSpec((bsz, _XP, 4), lambda i: (i, 0, 0)),
            _bcast((9, c)), _bcast((16, c, c)), _bcast((16, c, c)),
            _bcast((1, c)), _bcast((1, c)), _bcast((1, c)),
            _bcast((1, c)), _bcast((1, c)), _bcast((1, c)),
            _bcast((1, c)), _bcast((1, c)), _bcast((1, c)),
            _bcast((c, n_cls)), _bcast((1, n_cls)),
            _bcast((4, _P1, 1)), _bcast((_M2, 1)), _bcast((_M3, 1)),
            _bcast((c, c)), _bcast((c, c)), _bcast((c, c)),
        ],
        out_specs=pl.BlockSpec((bsz, n_cls), lambda i: (i, 0)),
        out_shape=jax.ShapeDtypeStruct((n, n_cls), jnp.float32),
        scratch_shapes=[
            pltpu.VMEM((bsz, 4, _P1, c), jnp.float32),
            pltpu.VMEM((bsz, 16, 16, c), jnp.float32),
        ],
        compiler_params=pltpu.CompilerParams(
            dimension_semantics=("parallel",)),
    )(xph, w1c, w2t, w3t, b1.reshape(1, c), b2.reshape(1, c),
      b3.reshape(1, c), gn1_g.reshape(1, c), gn1_b.reshape(1, c),
      gn2_g.reshape(1, c), gn2_b.reshape(1, c), gn3_g.reshape(1, c),
      gn3_b.reshape(1, c), fc_w.T, fc_b.reshape(1, n_cls), m1, m2, m3,
      a1, a2, a3)
    return out
